# baseline (speedup 1.0000x reference)
import functools

import jax
import jax.numpy as jnp
from jax.experimental import pallas as pl
from jax.experimental.pallas import tpu as pltpu


import numpy as np


def _band_const(rows, width):
    c = np.zeros((25, 5, rows, width), np.float32)
    for t in range(25):
        dy, dx = divmod(t, 5)
        for x in range(rows):
            c[t, dy, x, x + dx] = 1.0
    return c.reshape(25, 5 * rows * width)


def _fused_cnn_kernel(x_ref, w_ref, b_ref, o_ref, xt_ref, a1_ref, a2_ref, *, nb):
    f32 = jnp.float32
    bf16 = jnp.bfloat16

    eye = jnp.eye(128, dtype=bf16)
    for c in range(nb // 128):
        a = x_ref[c * 128:(c + 1) * 128, :].astype(bf16)
        t = jax.lax.dot_general(a, eye, (((0,), (0,)), ((), ())),
                                preferred_element_type=f32)
        xt_ref[:, :, :, c * 128:(c + 1) * 128] = (
            t.astype(bf16).reshape(3, 32, 32, 128))

    t1 = w_ref[0:224, 0:480]
    b1 = b_ref[0:224]
    for r in range(14):
        y0 = 2 * r
        s0 = xt_ref[:, y0:y0 + 5, :, :].reshape(480, nb)
        s1 = xt_ref[:, y0 + 1:y0 + 6, :, :].reshape(480, nb)
        o0 = jnp.dot(t1, s0, preferred_element_type=f32)
        o1 = jnp.dot(t1, s1, preferred_element_type=f32)
        m = jnp.maximum(o0, o1) + b1
        m = jnp.maximum(m, 0.0)
        v = m.reshape(14, 2, 8, nb)
        a1_ref[r] = jnp.maximum(v[:, 0], v[:, 1]).astype(bf16)

    t2 = w_ref[224:384, 0:560]
    b2 = b_ref[224:384]
    for ro in range(5):
        y0 = 2 * ro
        s0 = a1_ref[y0:y0 + 5].reshape(560, nb)
        s1 = a1_ref[y0 + 1:y0 + 6].reshape(560, nb)
        o0 = jnp.dot(t2, s0, preferred_element_type=f32)
        o1 = jnp.dot(t2, s1, preferred_element_type=f32)
        m = jnp.maximum(o0, o1) + b2
        m = jnp.maximum(m, 0.0)
        v = m.reshape(5, 2, 16, nb)
        a2_ref[ro] = jnp.maximum(v[:, 0], v[:, 1]).astype(bf16)

    z = a2_ref[...].reshape(400, nb)
    h = jnp.dot(w_ref[384:512, 0:400], z, preferred_element_type=f32) + b_ref[384:512]
    h = jnp.maximum(h, 0.0).astype(bf16)
    h = jnp.dot(w_ref[512:640, 0:128], h, preferred_element_type=f32) + b_ref[512:640]
    h = jnp.maximum(h, 0.0).astype(bf16)
    o_ref[...] = (jnp.dot(w_ref[640:656, 0:128], h, preferred_element_type=f32)
                  + b_ref[640:656])


def kernel(w1, b1, w2, b2, fc1_w, fc1_b, fc2_w, fc2_b, fc3_w, fc3_b, x):
    f32 = jnp.float32
    bf16 = jnp.bfloat16
    n = x.shape[0]
    nb = 1024 if n % 1024 == 0 else 128

    c1 = _band_const(28, 32)
    t1 = jax.lax.dot_general(w1[:, :3, :6].reshape(25, 18), c1,
                             (((0,), (0,)), ((), ())))
    t1 = t1.reshape(3, 6, 5, 28, 32)
    t1 = jnp.transpose(t1, (3, 1, 0, 2, 4))
    t1 = jnp.pad(t1, ((0, 0), (0, 2), (0, 0), (0, 0), (0, 0)))
    t1 = t1.reshape(224, 480)
    b1r = jnp.broadcast_to(jnp.pad(b1[0, :6], (0, 2))[None, :], (28, 8))
    b1r = b1r.reshape(224, 1)

    c2 = _band_const(10, 14)
    t2 = jax.lax.dot_general(w2[:, :6, :16].reshape(25, 96), c2,
                             (((0,), (0,)), ((), ())))
    t2 = t2.reshape(6, 16, 5, 10, 14)
    t2 = jnp.transpose(t2, (3, 1, 2, 4, 0))
    t2 = jnp.pad(t2, ((0, 0), (0, 0), (0, 0), (0, 0), (0, 2)))
    t2 = t2.reshape(160, 560)
    b2r = jnp.broadcast_to(b2[0, :16][None, :], (10, 16)).reshape(160, 1)

    f1 = fc1_w.reshape(5, 5, 128, 128)[:, :, :16, :120]
    w1b = jnp.transpose(f1, (3, 0, 1, 2)).reshape(120, 400)
    w1b = jnp.pad(w1b, ((0, 8), (0, 0)))
    w2b = fc2_w.T
    w3b = fc3_w.T[:16]

    def padw(a):
        return jnp.pad(a, ((0, 0), (0, 576 - a.shape[1])))

    wpack = jnp.concatenate(
        [padw(t1), padw(t2), padw(w1b), padw(w2b), padw(w3b)], axis=0
    ).astype(bf16)
    bpack = jnp.concatenate(
        [b1r, b2r, fc1_b.T, fc2_b.T, fc3_b[:, :16].T], axis=0
    ).astype(f32)

    x2 = x.reshape(n, 3 * 32 * 32)

    out = pl.pallas_call(
        functools.partial(_fused_cnn_kernel, nb=nb),
        out_shape=jax.ShapeDtypeStruct((16, n), f32),
        grid=(n // nb,),
        in_specs=[
            pl.BlockSpec((nb, 3 * 32 * 32), lambda i: (i, 0)),
            pl.BlockSpec(wpack.shape, lambda i: (0, 0)),
            pl.BlockSpec(bpack.shape, lambda i: (0, 0)),
        ],
        out_specs=pl.BlockSpec((16, nb), lambda i: (0, i)),
        scratch_shapes=[
            pltpu.VMEM((3, 32, 32, nb), bf16),
            pltpu.VMEM((14, 14, 8, nb), bf16),
            pltpu.VMEM((5, 5, 16, nb), bf16),
        ],
        compiler_params=pltpu.CompilerParams(
            dimension_semantics=("parallel",),
            fuse_transposed_lhs_in_matmul=True),
    )(x2, wpack, bpack)

    return out[:10, :].T


# speedup vs baseline: 1.3145x; 1.3145x over previous
import functools

import jax
import jax.numpy as jnp
import numpy as np
from jax.experimental import pallas as pl
from jax.experimental.pallas import tpu as pltpu


def _band_const(rows, width):
    c = np.zeros((25, 5, rows, width), np.float32)
    for t in range(25):
        dy, dx = divmod(t, 5)
        for x in range(rows):
            c[t, dy, x, x + dx] = 1.0
    return c.reshape(25, 5 * rows * width)


def _bias_const():
    s = np.zeros((656, 640), np.float32)
    for x in range(28):
        for co in range(6):
            s[x * 8 + co, co] = 1.0
    for x in range(10):
        for co in range(16):
            s[224 + x * 16 + co, 128 + co] = 1.0
    for o in range(128):
        s[384 + o, 256 + o] = 1.0
        s[512 + o, 384 + o] = 1.0
    for o in range(16):
        s[640 + o, 512 + o] = 1.0
    return s


def _fused_cnn_kernel(x_ref, t1_ref, t2_ref, w1_ref, w2_ref, w3_ref, b_ref,
                      o_ref, a1_ref, a2_ref, *, nb):
    f32 = jnp.float32
    bf16 = jnp.bfloat16

    t1 = t1_ref[...]
    b1 = b_ref[0:224]
    for r in range(14):
        y0 = 2 * r
        s0 = x_ref[:, y0:y0 + 5, :, :].reshape(480, nb)
        s1 = x_ref[:, y0 + 1:y0 + 6, :, :].reshape(480, nb)
        o0 = jnp.dot(t1, s0, preferred_element_type=f32)
        o1 = jnp.dot(t1, s1, preferred_element_type=f32)
        m = jnp.maximum(o0, o1) + b1
        m = jnp.maximum(m, 0.0)
        v = m.reshape(14, 2, 8, nb)
        a1_ref[r] = jnp.maximum(v[:, 0], v[:, 1]).astype(bf16)

    t2 = t2_ref[...]
    b2 = b_ref[224:384]
    for ro in range(5):
        y0 = 2 * ro
        s0 = a1_ref[y0:y0 + 5].reshape(560, nb)
        s1 = a1_ref[y0 + 1:y0 + 6].reshape(560, nb)
        o0 = jnp.dot(t2, s0, preferred_element_type=f32)
        o1 = jnp.dot(t2, s1, preferred_element_type=f32)
        m = jnp.maximum(o0, o1) + b2
        m = jnp.maximum(m, 0.0)
        v = m.reshape(5, 2, 16, nb)
        a2_ref[ro] = jnp.maximum(v[:, 0], v[:, 1]).astype(bf16)

    z = a2_ref[...].reshape(400, nb)
    h = jnp.dot(w1_ref[...], z, preferred_element_type=f32) + b_ref[384:512]
    h = jnp.maximum(h, 0.0).astype(bf16)
    h = jnp.dot(w2_ref[...], h, preferred_element_type=f32) + b_ref[512:640]
    h = jnp.maximum(h, 0.0).astype(bf16)
    o_ref[...] = (jnp.dot(w3_ref[...], h, preferred_element_type=f32)
                  + b_ref[640:656])


def kernel(w1, b1, w2, b2, fc1_w, fc1_b, fc2_w, fc2_b, fc3_w, fc3_b, x):
    f32 = jnp.float32
    bf16 = jnp.bfloat16
    n = x.shape[0]
    nb = 1024 if n % 1024 == 0 else 128

    c1 = _band_const(28, 32)
    t1 = jax.lax.dot_general(w1[:, :3, :6].reshape(25, 18), c1,
                             (((0,), (0,)), ((), ())))
    t1 = t1.reshape(3, 6, 5, 28, 32)
    t1 = jnp.transpose(t1, (3, 1, 0, 2, 4))
    t1 = jnp.pad(t1, ((0, 0), (0, 2), (0, 0), (0, 0), (0, 0)))
    t1 = t1.reshape(224, 480).astype(bf16)

    c2 = _band_const(10, 14)
    t2 = jax.lax.dot_general(w2[:, :6, :16].reshape(25, 96), c2,
                             (((0,), (0,)), ((), ())))
    t2 = t2.reshape(6, 16, 5, 10, 14)
    t2 = jnp.transpose(t2, (3, 1, 2, 4, 0))
    t2 = jnp.pad(t2, ((0, 0), (0, 0), (0, 0), (0, 0), (0, 2)))
    t2 = t2.reshape(160, 560).astype(bf16)

    f1 = fc1_w.reshape(5, 5, 128, 128)[:, :, :16, :120]
    w1b = jnp.transpose(f1, (3, 0, 1, 2)).reshape(120, 400)
    w1b = jnp.pad(w1b, ((0, 8), (0, 0))).astype(bf16)
    w2b = fc2_w.T.astype(bf16)
    w3b = fc3_w.T[:16].astype(bf16)

    braw = jnp.concatenate([b1, b2, fc1_b, fc2_b, fc3_b], axis=0)
    bpack = jnp.dot(_bias_const(), braw.reshape(640, 1))

    xt = jnp.transpose(x, (1, 2, 3, 0)).astype(bf16)

    def full(a):
        return pl.BlockSpec(a.shape, lambda i: (0,) * a.ndim)

    out = pl.pallas_call(
        functools.partial(_fused_cnn_kernel, nb=nb),
        out_shape=jax.ShapeDtypeStruct((16, n), f32),
        grid=(n // nb,),
        in_specs=[
            pl.BlockSpec((3, 32, 32, nb), lambda i: (0, 0, 0, i)),
            full(t1), full(t2), full(w1b), full(w2b), full(w3b), full(bpack),
        ],
        out_specs=pl.BlockSpec((16, nb), lambda i: (0, i)),
        scratch_shapes=[
            pltpu.VMEM((14, 14, 8, nb), bf16),
            pltpu.VMEM((5, 5, 16, nb), bf16),
        ],
        compiler_params=pltpu.CompilerParams(
            dimension_semantics=("parallel",)),
    )(xt, t1, t2, w1b, w2b, w3b, bpack)

    return out[:10, :].T
